# baseline (speedup 1.0000x reference)
import functools

import jax
import jax.numpy as jnp
import numpy as np
from jax import lax
from jax.experimental import pallas as pl
from jax.experimental.pallas import tpu as pltpu
from jax.experimental.pallas import tpu_sc as plsc

_FIELD_DIMS = [100000] * 26
F = 26
K = 16
B = 16384
D_IN = F * K
N = B * F
NW = 32
NPW = N // NW
CH = 832
NCH = NPW // CH

_OFFSETS = np.concatenate([[0], np.cumsum(_FIELD_DIMS[:-1])]).astype(np.int32)

_mesh = plsc.VectorSubcoreMesh(core_axis_name="c", subcore_axis_name="s")


@functools.partial(
    pl.kernel,
    out_type=(
        jax.ShapeDtypeStruct((NW, NPW, K), jnp.float32),
        jax.ShapeDtypeStruct((NW, NPW), jnp.float32),
    ),
    mesh=_mesh,
    scratch_types=[
        pltpu.VMEM((NPW,), jnp.int32),
        pltpu.VMEM((2, K, CH), jnp.float32),
        pltpu.VMEM((2, CH, K), jnp.float32),
        pltpu.VMEM((NPW,), jnp.float32),
        pltpu.SemaphoreType.DMA,
        pltpu.SemaphoreType.DMA,
        pltpu.SemaphoreType.DMA,
        pltpu.SemaphoreType.DMA,
        pltpu.SemaphoreType.DMA,
    ],
    compiler_params=pltpu.CompilerParams(
        use_tc_tiling_on_sc=False, needs_layout_passes=False),
)
def _sc_gather(idx_hbm, embt_hbm, fct_hbm, emb_out, fc_out,
               idx_v, gbuf, ebuf, fc_v, sem_a, sem_b, sem_w0, sem_w1, sem_fc):
    wid = lax.axis_index("s") * 2 + lax.axis_index("c")
    sems = (sem_a, sem_b)
    wsems = (sem_w0, sem_w1)

    pltpu.sync_copy(idx_hbm.at[wid], idx_v)
    fc_cp = pltpu.async_copy(fct_hbm.at[0].at[idx_v], fc_v, sem_fc)

    def start_chunk(j):
        cps = []
        for k in range(K):
            cps.append(pltpu.async_copy(
                embt_hbm.at[k].at[idx_v.at[pl.ds(j * CH, CH)]],
                gbuf.at[j % 2, k], sems[j % 2]))
        return cps

    lane = lax.broadcasted_iota(jnp.int32, (K,), 0)

    copies = [None, None]
    writes = [None, None]
    copies[0] = start_chunk(0)
    for j in range(NCH):
        p = j % 2
        if j + 1 < NCH:
            copies[1 - p] = start_chunk(j + 1)
        for cp in copies[p]:
            cp.wait()

        if writes[p] is not None:
            writes[p].wait()

        def body(n, _):
            row = plsc.load_gather(gbuf.at[p], [lane, jnp.full((K,), n, jnp.int32)])
            ebuf[p, n] = row
            return 0

        lax.fori_loop(0, CH, body, 0, unroll=8)
        writes[p] = pltpu.async_copy(
            ebuf.at[p], emb_out.at[wid, pl.ds(j * CH, CH)], wsems[p])

    for w in writes:
        if w is not None:
            w.wait()
    fc_cp.wait()
    pltpu.sync_copy(fc_v, fc_out.at[wid])


BT = 1024


def _tc_body(emb_ref, fc_ref, s_ref, w1_ref, b1_ref, w2_ref, b2_ref,
             w3_ref, bias_ref, out_ref):
    e = emb_ref[...]
    t = jnp.dot(e, s_ref[...], preferred_element_type=jnp.float32)
    fm = 0.5 * (jnp.sum(t * t, axis=1) - jnp.sum(e * e, axis=1))
    fcs = jnp.sum(fc_ref[...], axis=1)
    h = jnp.dot(e, w1_ref[...], preferred_element_type=jnp.float32)
    h = jnp.maximum(h + b1_ref[...], 0.0)
    h = jnp.dot(h, w2_ref[...], preferred_element_type=jnp.float32)
    h = jnp.maximum(h + b2_ref[...], 0.0)
    y = jnp.sum(h * w3_ref[...], axis=1)
    out_ref[...] = fm + fcs + y + bias_ref[0]


def kernel(x, emb_table, fc_table, bias, W1, b1, W2, b2, W3, b3):
    idx = (x + _OFFSETS[None, :]).reshape(NW, NPW)

    emb_rows, fc_vals = _sc_gather(idx, emb_table.T, fc_table.T)
    emb_flat = emb_rows.reshape(B, D_IN)
    fc_mat = fc_vals.reshape(B, F)

    s_pad = jnp.asarray(
        np.equal(np.arange(D_IN)[:, None] % K, np.arange(128)[None, :])
        .astype(np.float32))

    scores = pl.pallas_call(
        _tc_body,
        grid=(B // BT,),
        in_specs=[
            pl.BlockSpec((BT, D_IN), lambda i: (i, 0)),
            pl.BlockSpec((BT, F), lambda i: (i, 0)),
            pl.BlockSpec((D_IN, 128), lambda i: (0, 0)),
            pl.BlockSpec((D_IN, 256), lambda i: (0, 0)),
            pl.BlockSpec((1, 256), lambda i: (0, 0)),
            pl.BlockSpec((256, 128), lambda i: (0, 0)),
            pl.BlockSpec((1, 128), lambda i: (0, 0)),
            pl.BlockSpec((1, 128), lambda i: (0, 0)),
            pl.BlockSpec((1,), lambda i: (0,)),
        ],
        out_specs=pl.BlockSpec((BT,), lambda i: (i,)),
        out_shape=jax.ShapeDtypeStruct((B,), jnp.float32),
    )(emb_flat, fc_mat, s_pad, W1, b1.reshape(1, 256), W2,
      b2.reshape(1, 128), W3.reshape(1, 128), bias)
    return scores


# speedup vs baseline: 3.3389x; 3.3389x over previous
import functools

import jax
import jax.numpy as jnp
import numpy as np
from jax import lax
from jax.experimental import pallas as pl
from jax.experimental.pallas import tpu as pltpu
from jax.experimental.pallas import tpu_sc as plsc

_FIELD_DIMS = [100000] * 26
F = 26
K = 16
B = 16384
D_IN = F * K
N = B * F
NW = 32
NPW = N // NW
CH = 1664
NCH = NPW // CH
V = 2600000

DW = 1024
RPC = DW * K // 128
VMAIN = (V // 128) * 128
NFULL = VMAIN // DW
TPW = (NFULL + NW - 1) // NW

_OFFSETS = np.concatenate([[0], np.cumsum(_FIELD_DIMS[:-1])]).astype(np.int32)

_mesh = plsc.VectorSubcoreMesh(core_axis_name="c", subcore_axis_name="s")


@functools.partial(
    pl.kernel,
    out_type=jax.ShapeDtypeStruct((V * K // 128, 128), jnp.float32),
    mesh=_mesh,
    scratch_types=[
        pltpu.VMEM((2, 2, 8, DW), jnp.float32),
        pltpu.VMEM((2, RPC, 128), jnp.float32),
        pltpu.SemaphoreType.DMA,
        pltpu.SemaphoreType.DMA,
        pltpu.SemaphoreType.DMA,
        pltpu.SemaphoreType.DMA,
    ],
    compiler_params=pltpu.CompilerParams(needs_layout_passes=False),
)
def _sc_detile(embt_hbm, tail_hbm, tab_out, kbuf, rbuf,
               sem_r0, sem_r1, sem_w0, sem_w1):
    wid = lax.axis_index("s") * 2 + lax.axis_index("c")
    lane = lax.broadcasted_iota(jnp.int32, (K,), 0)
    nt = jnp.minimum(TPW, jnp.maximum(0, NFULL - wid * TPW))
    wsems = (sem_w0, sem_w1)

    def start_reads(t, p):
        c = wid * TPW + t
        j0 = c * DW
        for h in range(2):
            pltpu.async_copy(embt_hbm.at[pl.ds(h * 8, 8), pl.ds(j0, DW)],
                             kbuf.at[p, h], sem_r0)

    def wait_reads(p):
        for h in range(2):
            pltpu.make_async_copy(embt_hbm.at[pl.ds(h * 8, 8), pl.ds(0, DW)],
                                  kbuf.at[p, h], sem_r0).wait()

    lane_h = lane >> 3
    lane_r = lane & 7

    def transpose(p, nrows):
        def body(n, _):
            row = plsc.load_gather(
                kbuf.at[p], [lane_h, lane_r, jnp.full((K,), n, jnp.int32)])
            rbuf[p, n >> 3, pl.ds((n & 7) * K, K)] = row
            return 0
        lax.fori_loop(0, nrows, body, 0, unroll=8)

    def step(g, _):
        for b in (0, 1):
            t = 2 * g + b

            @pl.when(t < nt)
            def _(t=t, b=b):
                wait_reads(b)

                @pl.when(t + 1 < nt)
                def _():
                    start_reads(t + 1, 1 - b)

                @pl.when(t >= 2)
                def _():
                    pltpu.make_async_copy(rbuf.at[b], tab_out.at[pl.ds(0, RPC)],
                                          wsems[b]).wait()

                transpose(b, DW)
                c = wid * TPW + t
                pltpu.async_copy(rbuf.at[b], tab_out.at[pl.ds(c * RPC, RPC)],
                                 wsems[b])
        return 0

    start_reads(0, 0)
    lax.fori_loop(0, TPW // 2, step, 0)
    pltpu.make_async_copy(rbuf.at[0], tab_out.at[pl.ds(0, RPC)],
                          sem_w0).wait()
    pltpu.make_async_copy(rbuf.at[1], tab_out.at[pl.ds(0, RPC)],
                          sem_w1).wait()

    @pl.when(wid == NW - 1)
    def _():
        pltpu.sync_copy(tail_hbm, rbuf.at[0, pl.ds(0, 8)])
        pltpu.sync_copy(rbuf.at[0, pl.ds(0, 8)],
                        tab_out.at[pl.ds(NFULL * RPC, 8)])


@functools.partial(
    pl.kernel,
    out_type=(
        jax.ShapeDtypeStruct((NW, NPW, K), jnp.float32),
        jax.ShapeDtypeStruct((NW, NPW), jnp.float32),
    ),
    mesh=_mesh,
    scratch_types=[
        pltpu.VMEM((NPW,), jnp.int32),
        pltpu.VMEM((2, CH, K), jnp.float32),
        pltpu.VMEM((NPW,), jnp.float32),
        pltpu.SemaphoreType.DMA,
        pltpu.SemaphoreType.DMA,
        pltpu.SemaphoreType.DMA,
    ],
    compiler_params=pltpu.CompilerParams(use_tc_tiling_on_sc=False),
)
def _sc_gather(idx_hbm, tab_hbm, fc_hbm, emb_out, fc_out,
               idx_v, ebuf, fc_v, sem_a, sem_b, sem_fc):
    wid = lax.axis_index("s") * 2 + lax.axis_index("c")
    sems = (sem_a, sem_b)

    pltpu.sync_copy(idx_hbm.at[wid], idx_v)
    fc_cp = pltpu.async_copy(fc_hbm.at[0].at[idx_v], fc_v, sem_fc)

    copies = [None, None]
    copies[0] = pltpu.async_copy(
        tab_hbm.at[idx_v.at[pl.ds(0, CH)]], ebuf.at[0], sems[0])
    for j in range(NCH):
        if j + 1 < NCH:
            copies[(j + 1) % 2] = pltpu.async_copy(
                tab_hbm.at[idx_v.at[pl.ds((j + 1) * CH, CH)]],
                ebuf.at[(j + 1) % 2], sems[(j + 1) % 2])
        copies[j % 2].wait()
        pltpu.sync_copy(ebuf.at[j % 2], emb_out.at[wid, pl.ds(j * CH, CH)])

    fc_cp.wait()
    pltpu.sync_copy(fc_v, fc_out.at[wid])


BT = 1024


def _tc_body(emb_ref, fc_ref, s_ref, w1_ref, b1_ref, w2_ref, b2_ref,
             w3_ref, bias_ref, out_ref):
    e = emb_ref[...]
    t = jnp.dot(e, s_ref[...], preferred_element_type=jnp.float32)
    fm = 0.5 * (jnp.sum(t * t, axis=1) - jnp.sum(e * e, axis=1))
    fcs = jnp.sum(fc_ref[...], axis=1)
    h = jnp.dot(e, w1_ref[...], preferred_element_type=jnp.float32)
    h = jnp.maximum(h + b1_ref[...], 0.0)
    h = jnp.dot(h, w2_ref[...], preferred_element_type=jnp.float32)
    h = jnp.maximum(h + b2_ref[...], 0.0)
    y = jnp.sum(h * w3_ref[...], axis=1)
    out_ref[...] = fm + fcs + y + bias_ref[0]


def kernel(x, emb_table, fc_table, bias, W1, b1, W2, b2, W3, b3):
    idx = (x + _OFFSETS[None, :]).reshape(NW, NPW)

    tail_lin = emb_table[VMAIN:, :].reshape(8, 128)
    tab_lin = _sc_detile(emb_table.T, tail_lin).reshape(V, K)
    emb_rows, fc_vals = _sc_gather(idx, tab_lin, fc_table.T)
    emb_flat = emb_rows.reshape(B, D_IN)
    fc_mat = fc_vals.reshape(B, F)

    s_pad = jnp.asarray(
        np.equal(np.arange(D_IN)[:, None] % K, np.arange(128)[None, :])
        .astype(np.float32))

    scores = pl.pallas_call(
        _tc_body,
        grid=(B // BT,),
        in_specs=[
            pl.BlockSpec((BT, D_IN), lambda i: (i, 0)),
            pl.BlockSpec((BT, F), lambda i: (i, 0)),
            pl.BlockSpec((D_IN, 128), lambda i: (0, 0)),
            pl.BlockSpec((D_IN, 256), lambda i: (0, 0)),
            pl.BlockSpec((1, 256), lambda i: (0, 0)),
            pl.BlockSpec((256, 128), lambda i: (0, 0)),
            pl.BlockSpec((1, 128), lambda i: (0, 0)),
            pl.BlockSpec((1, 128), lambda i: (0, 0)),
            pl.BlockSpec((1,), lambda i: (0,)),
        ],
        out_specs=pl.BlockSpec((BT,), lambda i: (i,)),
        out_shape=jax.ShapeDtypeStruct((B,), jnp.float32),
    )(emb_flat, fc_mat, s_pad, W1, b1.reshape(1, 256), W2,
      b2.reshape(1, 128), W3.reshape(1, 128), bias)
    return scores


# speedup vs baseline: 8.6848x; 2.6011x over previous
import functools

import jax
import jax.numpy as jnp
import numpy as np
from jax import lax
from jax.experimental import pallas as pl
from jax.experimental.pallas import tpu as pltpu
from jax.experimental.pallas import tpu_sc as plsc

_FIELD_DIMS = [100000] * 26
F = 26
K = 16
B = 16384
D_IN = F * K
N = B * F
NW = 32
NPW = N // NW
CH = 1664
NCH = NPW // CH
V = 2600000

DW = 1024
RPC = DW * K // 128
VMAIN = (V // 128) * 128
NFULL = VMAIN // DW
TPW = (NFULL + NW - 1) // NW

_OFFSETS = np.concatenate([[0], np.cumsum(_FIELD_DIMS[:-1])]).astype(np.int32)

_mesh = plsc.VectorSubcoreMesh(core_axis_name="c", subcore_axis_name="s")


@functools.partial(
    pl.kernel,
    out_type=jax.ShapeDtypeStruct((V * K,), jnp.float32),
    mesh=_mesh,
    scratch_types=[
        pltpu.VMEM((2, 2, 8, DW), jnp.float32),
        pltpu.VMEM((DW * K,), jnp.float32),
        pltpu.VMEM((DW * K,), jnp.float32),
        pltpu.SemaphoreType.DMA,
        pltpu.SemaphoreType.DMA,
        pltpu.SemaphoreType.DMA,
        pltpu.SemaphoreType.DMA,
    ],
    compiler_params=pltpu.CompilerParams(needs_layout_passes=False),
)
def _sc_detile(embt_hbm, tail_hbm, tab_out, kbuf, rbuf0, rbuf1,
               sem_r0, sem_r1, sem_w0, sem_w1):
    wid = lax.axis_index("s") * 2 + lax.axis_index("c")
    lane = lax.broadcasted_iota(jnp.int32, (K,), 0)
    nt = jnp.minimum(TPW, jnp.maximum(0, NFULL - wid * TPW))
    wsems = (sem_w0, sem_w1)
    rbufs = (rbuf0, rbuf1)
    CSZ = DW * K
    base16 = lane * K

    def start_reads(t, p):
        c = wid * TPW + t
        j0 = c * DW
        for h in range(2):
            pltpu.async_copy(embt_hbm.at[pl.ds(h * 8, 8), pl.ds(j0, DW)],
                             kbuf.at[p, h], sem_r0)

    def wait_reads(p):
        for h in range(2):
            pltpu.make_async_copy(embt_hbm.at[pl.ds(h * 8, 8), pl.ds(0, DW)],
                                  kbuf.at[p, h], sem_r0).wait()

    def transpose(b):
        rb = rbufs[b]

        def body(m, _):
            n0 = m * K
            f0 = m * K * K
            for k in range(K):
                v = kbuf[b, k >> 3, k & 7, pl.ds(n0, K)]
                plsc.store_scatter(rb, [base16 + (f0 + k)], v)
            return 0

        lax.fori_loop(0, DW // K, body, 0, unroll=2)

    def step(g, _):
        for b in (0, 1):
            t = 2 * g + b

            @pl.when(t < nt)
            def _(t=t, b=b):
                wait_reads(b)

                @pl.when(t + 1 < nt)
                def _():
                    start_reads(t + 1, 1 - b)

                @pl.when(t >= 2)
                def _():
                    pltpu.make_async_copy(rbufs[b], tab_out.at[pl.ds(0, CSZ)],
                                          wsems[b]).wait()

                transpose(b)
                c = wid * TPW + t
                pltpu.async_copy(rbufs[b], tab_out.at[pl.ds(c * CSZ, CSZ)],
                                 wsems[b])
        return 0

    start_reads(0, 0)
    lax.fori_loop(0, TPW // 2, step, 0)
    pltpu.make_async_copy(rbuf0, tab_out.at[pl.ds(0, CSZ)], sem_w0).wait()
    pltpu.make_async_copy(rbuf1, tab_out.at[pl.ds(0, CSZ)], sem_w1).wait()

    @pl.when(wid == NW - 1)
    def _():
        pltpu.sync_copy(tail_hbm, rbuf0.at[pl.ds(0, 1024)])
        pltpu.sync_copy(rbuf0.at[pl.ds(0, 1024)],
                        tab_out.at[pl.ds(NFULL * CSZ, 1024)])


@functools.partial(
    pl.kernel,
    out_type=(
        jax.ShapeDtypeStruct((NW, NPW, K), jnp.float32),
        jax.ShapeDtypeStruct((NW, NPW), jnp.float32),
    ),
    mesh=_mesh,
    scratch_types=[
        pltpu.VMEM((NPW,), jnp.int32),
        pltpu.VMEM((2, CH, K), jnp.float32),
        pltpu.VMEM((NPW,), jnp.float32),
        pltpu.SemaphoreType.DMA,
        pltpu.SemaphoreType.DMA,
        pltpu.SemaphoreType.DMA,
    ],
    compiler_params=pltpu.CompilerParams(use_tc_tiling_on_sc=False),
)
def _sc_gather(idx_hbm, tab_hbm, fc_hbm, emb_out, fc_out,
               idx_v, ebuf, fc_v, sem_a, sem_b, sem_fc):
    wid = lax.axis_index("s") * 2 + lax.axis_index("c")
    sems = (sem_a, sem_b)

    pltpu.sync_copy(idx_hbm.at[wid], idx_v)
    fc_cp = pltpu.async_copy(fc_hbm.at[0].at[idx_v], fc_v, sem_fc)

    copies = [None, None]
    copies[0] = pltpu.async_copy(
        tab_hbm.at[idx_v.at[pl.ds(0, CH)]], ebuf.at[0], sems[0])
    for j in range(NCH):
        if j + 1 < NCH:
            copies[(j + 1) % 2] = pltpu.async_copy(
                tab_hbm.at[idx_v.at[pl.ds((j + 1) * CH, CH)]],
                ebuf.at[(j + 1) % 2], sems[(j + 1) % 2])
        copies[j % 2].wait()
        pltpu.sync_copy(ebuf.at[j % 2], emb_out.at[wid, pl.ds(j * CH, CH)])

    fc_cp.wait()
    pltpu.sync_copy(fc_v, fc_out.at[wid])


BT = 1024


def _tc_body(emb_ref, fc_ref, s_ref, w1_ref, b1_ref, w2_ref, b2_ref,
             w3_ref, bias_ref, out_ref):
    e = emb_ref[...]
    t = jnp.dot(e, s_ref[...], preferred_element_type=jnp.float32)
    fm = 0.5 * (jnp.sum(t * t, axis=1) - jnp.sum(e * e, axis=1))
    fcs = jnp.sum(fc_ref[...], axis=1)
    h = jnp.dot(e, w1_ref[...], preferred_element_type=jnp.float32)
    h = jnp.maximum(h + b1_ref[...], 0.0)
    h = jnp.dot(h, w2_ref[...], preferred_element_type=jnp.float32)
    h = jnp.maximum(h + b2_ref[...], 0.0)
    y = jnp.sum(h * w3_ref[...], axis=1)
    out_ref[...] = fm + fcs + y + bias_ref[0]


def kernel(x, emb_table, fc_table, bias, W1, b1, W2, b2, W3, b3):
    idx = (x + _OFFSETS[None, :]).reshape(NW, NPW)

    tail_lin = emb_table[VMAIN:, :].reshape(1024)
    tab_lin = _sc_detile(emb_table.T, tail_lin).reshape(V, K)
    emb_rows, fc_vals = _sc_gather(idx, tab_lin, fc_table.T)
    emb_flat = emb_rows.reshape(B, D_IN)
    fc_mat = fc_vals.reshape(B, F)

    s_pad = jnp.asarray(
        np.equal(np.arange(D_IN)[:, None] % K, np.arange(128)[None, :])
        .astype(np.float32))

    scores = pl.pallas_call(
        _tc_body,
        grid=(B // BT,),
        in_specs=[
            pl.BlockSpec((BT, D_IN), lambda i: (i, 0)),
            pl.BlockSpec((BT, F), lambda i: (i, 0)),
            pl.BlockSpec((D_IN, 128), lambda i: (0, 0)),
            pl.BlockSpec((D_IN, 256), lambda i: (0, 0)),
            pl.BlockSpec((1, 256), lambda i: (0, 0)),
            pl.BlockSpec((256, 128), lambda i: (0, 0)),
            pl.BlockSpec((1, 128), lambda i: (0, 0)),
            pl.BlockSpec((1, 128), lambda i: (0, 0)),
            pl.BlockSpec((1,), lambda i: (0,)),
        ],
        out_specs=pl.BlockSpec((BT,), lambda i: (i,)),
        out_shape=jax.ShapeDtypeStruct((B,), jnp.float32),
    )(emb_flat, fc_mat, s_pad, W1, b1.reshape(1, 256), W2,
      b2.reshape(1, 128), W3.reshape(1, 128), bias)
    return scores
